# initial kernel scaffold (unmeasured)
import jax
import jax.numpy as jnp
from jax import lax
from jax.experimental import pallas as pl
from jax.experimental.pallas import tpu as pltpu

N_DEV = 8


def kernel(x, W):
    m, _ = x.shape
    n_per = W.shape[1]
    half_n = n_per // 2

    xb = x.astype(jnp.bfloat16)
    Wb = W.astype(jnp.bfloat16)
    logits = jnp.dot(xb, Wb, preferred_element_type=jnp.float32)
    e32 = jnp.exp(logits)
    e = e32.astype(jnp.bfloat16)
    s = jnp.broadcast_to(
        e32.sum(axis=1, keepdims=True), (m, 128)
    )

    def body(e_ref, s_ref, out_ref, comm_ref, stats_ref, stage_ref,
             dsend, drecv, ssend, srecv, osem, credit):
        my = lax.axis_index("i")
        left = lax.rem(my + N_DEV - 1, N_DEV)
        right = lax.rem(my + 1, N_DEV)

        bar = pltpu.get_barrier_semaphore()
        for off in range(1, N_DEV):
            pl.semaphore_signal(
                bar, inc=1,
                device_id=(lax.rem(my + off, N_DEV),),
                device_id_type=pl.DeviceIdType.MESH,
            )
        pl.semaphore_wait(bar, N_DEV - 1)

        def data_rdma(h):
            src = e_ref if h == 0 else comm_ref.at[(h - 1) % 2]
            return pltpu.make_async_remote_copy(
                src_ref=src,
                dst_ref=comm_ref.at[h % 2],
                send_sem=dsend.at[h],
                recv_sem=drecv.at[h],
                device_id=(right,),
                device_id_type=pl.DeviceIdType.MESH,
            )

        data_rdma(0).start()

        stat_rdmas = []
        for off in range(1, N_DEV):
            r = pltpu.make_async_remote_copy(
                src_ref=s_ref,
                dst_ref=stats_ref.at[off - 1],
                send_sem=ssend.at[off - 1],
                recv_sem=srecv.at[off - 1],
                device_id=(lax.rem(my + off, N_DEV),),
                device_id_type=pl.DeviceIdType.MESH,
            )
            r.start()
            stat_rdmas.append(r)
        total = s_ref[:, :]
        for off in range(1, N_DEV):
            stat_rdmas[off - 1].wait_recv()
            total = total + stats_ref[off - 1]
        inv = 1.0 / total[:, 0:1]
        for off in range(1, N_DEV):
            stat_rdmas[off - 1].wait_send()

        state = {"n": 0}
        pending = []

        def store_half(vals, col_off):
            i = state["n"]
            sslot = i % 2
            if i >= 2:
                pending[i - 2].wait()
            stage_ref[sslot] = vals
            cp = pltpu.make_async_copy(
                stage_ref.at[sslot],
                out_ref.at[:, pl.ds(col_off, half_n)],
                osem.at[sslot],
            )
            cp.start()
            pending.append(cp)
            state["n"] = i + 1

        for half in range(2):
            vals = e_ref[:, half * half_n:(half + 1) * half_n].astype(
                jnp.float32) * inv
            store_half(vals, my * n_per + half * half_n)

        for h in range(N_DEV - 1):
            rd = data_rdma(h)
            rd.wait_recv()
            rd.wait_send()
            if 1 <= h <= N_DEV - 3:
                pl.semaphore_signal(
                    credit, inc=1,
                    device_id=(left,),
                    device_id_type=pl.DeviceIdType.MESH,
                )
            if h <= N_DEV - 3:
                if h + 1 >= 2:
                    pl.semaphore_wait(credit, 1)
                data_rdma(h + 1).start()
            origin = lax.rem(my + (N_DEV - 1 - h), N_DEV)
            for half in range(2):
                vals = comm_ref[
                    h % 2, :, half * half_n:(half + 1) * half_n
                ].astype(jnp.float32) * inv
                store_half(vals, origin * n_per + half * half_n)

        pending[-2].wait()
        pending[-1].wait()

    out_shape = jax.ShapeDtypeStruct((m, N_DEV * n_per), jnp.float32)
    return pl.pallas_call(
        body,
        out_shape=out_shape,
        in_specs=[
            pl.BlockSpec(memory_space=pltpu.VMEM),
            pl.BlockSpec(memory_space=pltpu.VMEM),
        ],
        out_specs=pl.BlockSpec(memory_space=pltpu.ANY),
        scratch_shapes=[
            pltpu.VMEM((2, m, n_per), jnp.bfloat16),
            pltpu.VMEM((N_DEV - 1, m, 128), jnp.float32),
            pltpu.VMEM((2, m, half_n), jnp.float32),
            pltpu.SemaphoreType.DMA((N_DEV - 1,)),
            pltpu.SemaphoreType.DMA((N_DEV - 1,)),
            pltpu.SemaphoreType.DMA((N_DEV - 1,)),
            pltpu.SemaphoreType.DMA((N_DEV - 1,)),
            pltpu.SemaphoreType.DMA((2,)),
            pltpu.SemaphoreType.REGULAR,
        ],
        compiler_params=pltpu.CompilerParams(collective_id=0),
    )(e, s)


# baseline (device time: 786685 ns/iter reference)
import jax
import jax.numpy as jnp
from jax import lax
from jax.experimental import pallas as pl
from jax.experimental.pallas import tpu as pltpu

N_DEV = 8


def kernel(x, W):
    m, _ = x.shape
    n_per = W.shape[1]
    half_n = n_per // 2

    xb = x.astype(jnp.bfloat16)
    Wb = W.astype(jnp.bfloat16)
    logits = jnp.dot(xb, Wb, preferred_element_type=jnp.float32)
    e32 = jnp.exp(logits)
    e = e32.astype(jnp.bfloat16)
    s = jnp.broadcast_to(
        e32.sum(axis=1, keepdims=True), (m, 128)
    )

    def body(e_ref, s_ref, out_ref, comm_ref, stats_ref, stage_ref,
             dsend, drecv, ssend, srecv, osem, credit):
        my = lax.axis_index("i")
        left = lax.rem(my + N_DEV - 1, N_DEV)
        right = lax.rem(my + 1, N_DEV)

        bar = pltpu.get_barrier_semaphore()
        for off in range(1, N_DEV):
            pl.semaphore_signal(
                bar, inc=1,
                device_id=(lax.rem(my + off, N_DEV),),
                device_id_type=pl.DeviceIdType.MESH,
            )
        pl.semaphore_wait(bar, N_DEV - 1)

        def data_rdma(h):
            src = e_ref if h == 0 else comm_ref.at[(h - 1) % 2]
            return pltpu.make_async_remote_copy(
                src_ref=src,
                dst_ref=comm_ref.at[h % 2],
                send_sem=dsend.at[h],
                recv_sem=drecv.at[h],
                device_id=(right,),
                device_id_type=pl.DeviceIdType.MESH,
            )

        data_rdma(0).start()

        stat_rdmas = []
        for off in range(1, N_DEV):
            r = pltpu.make_async_remote_copy(
                src_ref=s_ref,
                dst_ref=stats_ref.at[off - 1],
                send_sem=ssend.at[off - 1],
                recv_sem=srecv.at[off - 1],
                device_id=(lax.rem(my + off, N_DEV),),
                device_id_type=pl.DeviceIdType.MESH,
            )
            r.start()
            stat_rdmas.append(r)
        total = s_ref[:, :]
        for off in range(1, N_DEV):
            stat_rdmas[off - 1].wait_recv()
            total = total + stats_ref[off - 1]
        inv = 1.0 / total[:, 0:1]
        for off in range(1, N_DEV):
            stat_rdmas[off - 1].wait_send()

        state = {"n": 0}
        pending = []

        def store_half(vals, col_off):
            i = state["n"]
            sslot = i % 2
            if i >= 2:
                pending[i - 2].wait()
            stage_ref[sslot] = vals
            cp = pltpu.make_async_copy(
                stage_ref.at[sslot],
                out_ref.at[:, pl.ds(col_off, half_n)],
                osem.at[sslot],
            )
            cp.start()
            pending.append(cp)
            state["n"] = i + 1

        for half in range(2):
            vals = e_ref[:, half * half_n:(half + 1) * half_n].astype(
                jnp.float32) * inv
            store_half(vals, my * n_per + half * half_n)

        for h in range(N_DEV - 1):
            rd = data_rdma(h)
            rd.wait_recv()
            rd.wait_send()
            if 1 <= h <= N_DEV - 3:
                pl.semaphore_signal(
                    credit, inc=1,
                    device_id=(left,),
                    device_id_type=pl.DeviceIdType.MESH,
                )
            if h <= N_DEV - 3:
                if h + 1 >= 2:
                    pl.semaphore_wait(credit, 1)
                data_rdma(h + 1).start()
            origin = lax.rem(my + (N_DEV - 1 - h), N_DEV)
            for half in range(2):
                vals = comm_ref[
                    h % 2, :, half * half_n:(half + 1) * half_n
                ].astype(jnp.float32) * inv
                store_half(vals, origin * n_per + half * half_n)

        pending[-2].wait()
        pending[-1].wait()

    out_shape = jax.ShapeDtypeStruct((m, N_DEV * n_per), jnp.float32)
    return pl.pallas_call(
        body,
        out_shape=out_shape,
        in_specs=[
            pl.BlockSpec(memory_space=pltpu.VMEM),
            pl.BlockSpec(memory_space=pltpu.VMEM),
        ],
        out_specs=pl.BlockSpec(memory_space=pl.ANY),
        scratch_shapes=[
            pltpu.VMEM((2, m, n_per), jnp.bfloat16),
            pltpu.VMEM((N_DEV - 1, m, 128), jnp.float32),
            pltpu.VMEM((2, m, half_n), jnp.float32),
            pltpu.SemaphoreType.DMA((N_DEV - 1,)),
            pltpu.SemaphoreType.DMA((N_DEV - 1,)),
            pltpu.SemaphoreType.DMA((N_DEV - 1,)),
            pltpu.SemaphoreType.DMA((N_DEV - 1,)),
            pltpu.SemaphoreType.DMA((2,)),
            pltpu.SemaphoreType.REGULAR,
        ],
        compiler_params=pltpu.CompilerParams(
            collective_id=0,
            vmem_limit_bytes=60 * 1024 * 1024,
        ),
    )(e, s)
